# baseline (speedup 1.0000x reference)
import functools

import jax
import jax.numpy as jnp
from jax import lax
from jax.experimental import pallas as pl
from jax.experimental.pallas import tpu as pltpu
from jax.experimental.pallas import tpu_sc as plsc

_N = 10000
_E = 320000
_G = 64
_NSHAPES = 32
_F = 128
_H1 = 64
_H2 = 128
_DE = 16

_NTILES = 32
_EPW = _E // _NTILES
_K = 40
_NCH = _EPW // _K
_NP = 10240
_NPT = _NP // 16


def _tc_node_body(x_ref, emb_ref, wl_ref, bl_ref, wr_ref, br_ref,
                  xl_ref, xr_ref):
    xv = x_ref[...]
    col = lax.broadcasted_iota(jnp.int32, xv.shape, 1)
    rowmax = jnp.max(xv, axis=1, keepdims=True)
    idx = jnp.min(jnp.where(xv >= rowmax, col, 10 ** 9), axis=1, keepdims=True)
    onehot = (col == idx).astype(jnp.float32)
    nf = jnp.dot(onehot, emb_ref[...], preferred_element_type=jnp.float32)
    xl = jnp.dot(nf, wl_ref[...],
                 preferred_element_type=jnp.float32) + bl_ref[...]
    xl_ref[...] = _pack_cols(xl)
    xr = jnp.dot(nf, wr_ref[...],
                 preferred_element_type=jnp.float32) + br_ref[...]
    xr_ref[...] = _pack_cols(xr)


def _pack_cols(x):
    h2 = x.shape[1] // 2
    lo = lax.bitcast_convert_type(x[:, :h2].astype(jnp.bfloat16),
                                  jnp.int16).astype(jnp.int32) & 0xFFFF
    hi = lax.bitcast_convert_type(x[:, h2:].astype(jnp.bfloat16),
                                  jnp.int16).astype(jnp.int32)
    return lo | (hi << 16)


def _tc_node(x, emb_table, W_l1, b_l1, W_r1, b_r1):
    return pl.pallas_call(
        _tc_node_body,
        out_shape=(jax.ShapeDtypeStruct((_N, _H1 // 2), jnp.int32),
                   jax.ShapeDtypeStruct((_N, _H1 // 2), jnp.int32)),
    )(x, emb_table, W_l1, b_l1, W_r1, b_r1)


_BE = 16000


def _tc_em_body(ea_ref, we1_ref, we2_ref, em1_ref, em2_ref):
    ea = ea_ref[...]
    em1_ref[...] = _pack_cols(
        jnp.dot(ea, we1_ref[...], preferred_element_type=jnp.float32))
    em2_ref[...] = _pack_cols(
        jnp.dot(ea, we2_ref[...], preferred_element_type=jnp.float32))


def _tc_em(edge_attr, W_e1, W_e2):
    grid = (_E // _BE,)
    return pl.pallas_call(
        _tc_em_body,
        grid=grid,
        in_specs=[
            pl.BlockSpec((_BE, _DE), lambda i: (i, 0)),
            pl.BlockSpec((_DE, _H1), lambda i: (0, 0)),
            pl.BlockSpec((_DE, _H2), lambda i: (0, 0)),
        ],
        out_specs=(
            pl.BlockSpec((_BE, _H1 // 2), lambda i: (i, 0)),
            pl.BlockSpec((_BE, _H2 // 2), lambda i: (i, 0)),
        ),
        out_shape=(jax.ShapeDtypeStruct((_E, _H1 // 2), jnp.int32),
                   jax.ShapeDtypeStruct((_E, _H2 // 2), jnp.int32)),
    )(edge_attr, W_e1, W_e2)


_U = 5
_HIMASK = -65536
_KH = _K // 2


def _sc_edge_pass(H):
    HP = H + 16
    NG = H // 16

    mesh = plsc.VectorSubcoreMesh(core_axis_name="c", subcore_axis_name="s")

    @functools.partial(
        pl.kernel,
        out_type=jax.ShapeDtypeStruct((2 * _NP, HP), jnp.float32),
        mesh=mesh,
        compiler_params=pltpu.CompilerParams(needs_layout_passes=False,
                                             use_tc_tiling_on_sc=False),
        scratch_types=[
            pltpu.VMEM((2, _K), jnp.int32),
            pltpu.VMEM((2, 2, _KH), jnp.int32),
            pltpu.VMEM((2, _K, H // 2), jnp.int32),
            pltpu.VMEM((2, _K, H // 2), jnp.int32),
            pltpu.VMEM((2, _K, H // 2), jnp.int32),
            pltpu.VMEM((2, _KH, HP), jnp.float32),
            pltpu.VMEM((H,), jnp.float32),
            pltpu.VMEM_SHARED((_NP, HP), jnp.float32),
            pltpu.SemaphoreType.DMA,
            pltpu.SemaphoreType.DMA,
            pltpu.SemaphoreType.DMA,
            pltpu.SemaphoreType.DMA,
            pltpu.SemaphoreType.DMA,
            pltpu.SemaphoreType.DMA,
            pltpu.SemaphoreType.DMA,
            pltpu.SemaphoreType.DMA,
        ],
    )
    def k(xl_hbm, xr_hbm, em_hbm, src_hbm, dst_hbm, att_hbm, out_hbm,
          src_v, dst_v, rows_l, rows_r, em_v, wout, att_v, acc_sh,
          sl0, sr0, se0, sl1, sr1, se1, ss0, ss1):
        c = lax.axis_index("c")
        s = lax.axis_index("s")
        wid = c * 16 + s
        sems = ((sl0, sr0, se0), (sl1, sr1, se1))
        ssem = (ss0, ss1)

        pltpu.sync_copy(att_hbm, att_v)

        zero16 = jnp.zeros((16,), jnp.float32)

        def zw(e, carry):
            for h in range(2):
                for j in range(HP // 16):
                    wout[h, e, pl.ds(j * 16, 16)] = zero16
            return carry

        lax.fori_loop(0, _KH, zw, 0)
        for t in range(_NPT // _KH):
            pltpu.sync_copy(wout.at[t % 2],
                            acc_sh.at[pl.ds(s * _NPT + t * _KH, _KH)])
        plsc.subcore_barrier()

        lane = lax.iota(jnp.int32, 16)

        def issue(ch, b):
            base = wid * _EPW + ch * _K
            pltpu.sync_copy(src_hbm.at[pl.ds(base, _K)], src_v.at[b])
            pltpu.sync_copy(dst_hbm.at[pl.ds(base // _KH, 2)], dst_v.at[b])
            pltpu.async_copy(xl_hbm.at[src_v.at[b]], rows_l.at[b], sems[b][0])
            for h in range(2):
                pltpu.async_copy(xr_hbm.at[dst_v.at[b, h]],
                                 rows_r.at[b, pl.ds(h * _KH, _KH)],
                                 sems[b][1])
            pltpu.async_copy(em_hbm.at[pl.ds(base, _K)], em_v.at[b],
                             sems[b][2])

        def drain(b):
            pltpu.make_async_copy(xl_hbm.at[src_v.at[b]], rows_l.at[b],
                                  sems[b][0]).wait()
            for h in range(2):
                pltpu.make_async_copy(xr_hbm.at[dst_v.at[b, h]],
                                      rows_r.at[b, pl.ds(h * _KH, _KH)],
                                      sems[b][1]).wait()
            pltpu.make_async_copy(em_hbm.at[pl.ds(0, _K)], em_v.at[b],
                                  sems[b][2]).wait()

        def scat_wait(b, h):
            pltpu.make_async_copy(wout.at[h],
                                  acc_sh.at[dst_v.at[b, h]],
                                  ssem[h]).wait()

        def compute_half(b, h, att_regs):
            rl = rows_l.at[b]
            rr = rows_r.at[b]
            ev = em_v.at[b]

            def group(g, regs):
                w0 = g * _U
                for u in range(_U):
                    e = h * _KH + w0 + u
                    w = w0 + u
                    acc = None
                    lvs = [None] * NG
                    for q in range(NG // 2):
                        sl = pl.ds(q * 16, 16)
                        wl = rl[e, sl]
                        wr = rr[e, sl]
                        we = ev[e, sl]
                        lo = (plsc.bitcast(wr << 16, jnp.float32)
                              + plsc.bitcast(we << 16, jnp.float32))
                        hi = (plsc.bitcast(wr & _HIMASK, jnp.float32)
                              + plsc.bitcast(we & _HIMASK, jnp.float32))
                        lvs[q] = plsc.bitcast(wl << 16, jnp.float32)
                        lvs[NG // 2 + q] = plsc.bitcast(wl & _HIMASK,
                                                        jnp.float32)
                        for j, addv in ((q, lo), (NG // 2 + q, hi)):
                            mv = lvs[j] + addv
                            mv = jnp.maximum(mv, 0.2 * mv)
                            t = mv * regs[j]
                            acc = t if acc is None else acc + t
                    a = jnp.sum(acc)
                    aev = jnp.exp(jnp.broadcast_to(a, (16,)))
                    for j in range(NG):
                        wout[h, w, pl.ds(j * 16, 16)] = lvs[j] * aev
                    wout[h, w, pl.ds(H, 16)] = jnp.where(lane == 0, aev, 0.0)
                return regs

            return lax.fori_loop(0, _KH // _U, group, att_regs)

        att_regs0 = tuple(att_v[pl.ds(j * 16, 16)] for j in range(NG))

        issue(0, 0)

        def outer(i, att_regs):
            for b in (0, 1):
                ch = i * 2 + b
                drain(b)

                @pl.when(ch + 1 < _NCH)
                def _():
                    issue(ch + 1, 1 - b)

                for h in (0, 1):
                    @pl.when(ch >= 1)
                    def _():
                        scat_wait(1 - b, h)

                    att_regs = compute_half(b, h, att_regs)
                    pltpu.async_copy(wout.at[h],
                                     acc_sh.at[dst_v.at[b, h]],
                                     ssem[h], add=True)
            return att_regs

        lax.fori_loop(0, _NCH // 2, outer, att_regs0)
        for h in (0, 1):
            scat_wait(1, h)
        plsc.subcore_barrier()

        pltpu.sync_copy(acc_sh.at[pl.ds(s * _NPT, _NPT)],
                        out_hbm.at[pl.ds(c * _NP + s * _NPT, _NPT)])

    return k


_sc_edge_h1 = _sc_edge_pass(_H1)
_sc_edge_h2 = _sc_edge_pass(_H2)


def _tc_mid_body(acc_ref, bias_ref, gamma_ref, beta_ref,
                 wl_ref, bl_ref, wr_ref, br_ref, xl_ref, xr_ref):
    accv = acc_ref[...]
    tot = accv[0:_N, :] + accv[_NP:_NP + _N, :]
    asum = tot[:, _H1:_H1 + 1]
    d = tot[:, 0:_H1] / (asum + 1e-16) + bias_ref[...]
    mu = jnp.mean(d, axis=0, keepdims=True)
    var = jnp.mean((d - mu) ** 2, axis=0, keepdims=True)
    dn = (d - mu) / jnp.sqrt(var + 1e-5) * gamma_ref[...] + beta_ref[...]
    d1 = jnp.maximum(dn, 0.01 * dn)
    xl = jnp.dot(d1, wl_ref[...],
                 preferred_element_type=jnp.float32) + bl_ref[...]
    xl_ref[...] = _pack_cols(xl)
    xr = jnp.dot(d1, wr_ref[...],
                 preferred_element_type=jnp.float32) + br_ref[...]
    xr_ref[...] = _pack_cols(xr)


def _tc_mid(acc1, bias1, gamma1, beta1, W_l2, b_l2, W_r2, b_r2):
    return pl.pallas_call(
        _tc_mid_body,
        out_shape=(jax.ShapeDtypeStruct((_N, _H2 // 2), jnp.int32),
                   jax.ShapeDtypeStruct((_N, _H2 // 2), jnp.int32)),
    )(acc1, bias1, gamma1, beta1, W_l2, b_l2, W_r2, b_r2)


def _tc_final_body(acc_ref, bias_ref, gamma_ref, beta_ref,
                   gw_ref, gb_ref, batch_ref, out_ref):
    accv = acc_ref[...]
    tot = accv[0:_N, :] + accv[_NP:_NP + _N, :]
    asum = tot[:, _H2:_H2 + 1]
    d = tot[:, 0:_H2] / (asum + 1e-16) + bias_ref[...]
    mu = jnp.mean(d, axis=0, keepdims=True)
    var = jnp.mean((d - mu) ** 2, axis=0, keepdims=True)
    dn = (d - mu) / jnp.sqrt(var + 1e-5) * gamma_ref[...] + beta_ref[...]
    d2 = jnp.maximum(dn, 0.01 * dn)
    gate = jnp.dot(d2, gw_ref[...],
                   preferred_element_type=jnp.float32) + gb_ref[...]
    ge = jnp.exp(gate)
    oh = (lax.broadcasted_iota(jnp.int32, (_G, _N), 0)
          == batch_ref[...]).astype(jnp.float32)
    wsum = jnp.dot(oh, d2 * ge, preferred_element_type=jnp.float32)
    gs = jnp.dot(oh, ge, preferred_element_type=jnp.float32)
    out_ref[...] = wsum / (gs + 1e-16)


def _tc_final(acc2, bias2, gamma2, beta2, gate_W, gate_b, batch_row):
    return pl.pallas_call(
        _tc_final_body,
        out_shape=jax.ShapeDtypeStruct((_G, _H2), jnp.float32),
    )(acc2, bias2, gamma2, beta2, gate_W, gate_b, batch_row)


def kernel(x, edge_index, edge_attr, batch, emb_table, W_l1, b_l1, W_r1, b_r1,
           W_e1, att1, bias1, gamma1, beta1, W_l2, b_l2, W_r2, b_r2, W_e2,
           att2, bias2, gamma2, beta2, gate_W, gate_b):
    src = edge_index[0]
    dst = edge_index[1].reshape(_E // _KH, _KH)

    xl1, xr1 = _tc_node(x, emb_table,
                        W_l1, b_l1.reshape(1, _H1), W_r1, b_r1.reshape(1, _H1))
    em1, em2 = _tc_em(edge_attr, W_e1, W_e2)

    acc1 = _sc_edge_h1(xl1, xr1, em1, src, dst, att1)
    xl2, xr2 = _tc_mid(acc1, bias1.reshape(1, _H1), gamma1.reshape(1, _H1),
                       beta1.reshape(1, _H1), W_l2, b_l2.reshape(1, _H2),
                       W_r2, b_r2.reshape(1, _H2))

    acc2 = _sc_edge_h2(xl2, xr2, em2, src, dst, att2)
    out = _tc_final(acc2, bias2.reshape(1, _H2), gamma2.reshape(1, _H2),
                    beta2.reshape(1, _H2), gate_W, gate_b.reshape(1, 1),
                    batch.reshape(1, _N))
    return out


# speedup vs baseline: 1.0121x; 1.0121x over previous
import functools

import jax
import jax.numpy as jnp
from jax import lax
from jax.experimental import pallas as pl
from jax.experimental.pallas import tpu as pltpu
from jax.experimental.pallas import tpu_sc as plsc

_N = 10000
_E = 320000
_G = 64
_NSHAPES = 32
_F = 128
_H1 = 64
_H2 = 128
_DE = 16

_NTILES = 32
_EPW = _E // _NTILES
_K = 40
_NCH = _EPW // _K
_NP = 10240
_NPT = _NP // 16


def _tc_node_body(x_ref, emb_ref, wl_ref, bl_ref, wr_ref, br_ref,
                  xl_ref, xr_ref):
    xv = x_ref[...]
    col = lax.broadcasted_iota(jnp.int32, xv.shape, 1)
    rowmax = jnp.max(xv, axis=1, keepdims=True)
    idx = jnp.min(jnp.where(xv >= rowmax, col, 10 ** 9), axis=1, keepdims=True)
    onehot = (col == idx).astype(jnp.float32)
    nf = jnp.dot(onehot, emb_ref[...], preferred_element_type=jnp.float32)
    xl_ref[...] = jnp.dot(nf, wl_ref[...],
                          preferred_element_type=jnp.float32) + bl_ref[...]
    xr = jnp.dot(nf, wr_ref[...],
                 preferred_element_type=jnp.float32) + br_ref[...]
    xr_ref[...] = _pack_cols(xr)


def _pack_cols(x):
    h2 = x.shape[1] // 2
    lo = lax.bitcast_convert_type(x[:, :h2].astype(jnp.bfloat16),
                                  jnp.int16).astype(jnp.int32) & 0xFFFF
    hi = lax.bitcast_convert_type(x[:, h2:].astype(jnp.bfloat16),
                                  jnp.int16).astype(jnp.int32)
    return lo | (hi << 16)


def _tc_node(x, emb_table, W_l1, b_l1, W_r1, b_r1):
    return pl.pallas_call(
        _tc_node_body,
        out_shape=(jax.ShapeDtypeStruct((_N, _H1), jnp.float32),
                   jax.ShapeDtypeStruct((_N, _H1 // 2), jnp.int32)),
    )(x, emb_table, W_l1, b_l1, W_r1, b_r1)


_BE = 16000


def _tc_em_body(ea_ref, we1_ref, we2_ref, em1_ref, em2_ref):
    ea = ea_ref[...]
    em1_ref[...] = _pack_cols(
        jnp.dot(ea, we1_ref[...], preferred_element_type=jnp.float32))
    em2_ref[...] = _pack_cols(
        jnp.dot(ea, we2_ref[...], preferred_element_type=jnp.float32))


def _tc_em(edge_attr, W_e1, W_e2):
    grid = (_E // _BE,)
    return pl.pallas_call(
        _tc_em_body,
        grid=grid,
        in_specs=[
            pl.BlockSpec((_BE, _DE), lambda i: (i, 0)),
            pl.BlockSpec((_DE, _H1), lambda i: (0, 0)),
            pl.BlockSpec((_DE, _H2), lambda i: (0, 0)),
        ],
        out_specs=(
            pl.BlockSpec((_BE, _H1 // 2), lambda i: (i, 0)),
            pl.BlockSpec((_BE, _H2 // 2), lambda i: (i, 0)),
        ),
        out_shape=(jax.ShapeDtypeStruct((_E, _H1 // 2), jnp.int32),
                   jax.ShapeDtypeStruct((_E, _H2 // 2), jnp.int32)),
    )(edge_attr, W_e1, W_e2)


_U = 5
_HIMASK = -65536
_KH = _K // 2


def _sc_edge_pass(H):
    HP = H + 16
    NG = H // 16

    mesh = plsc.VectorSubcoreMesh(core_axis_name="c", subcore_axis_name="s")

    @functools.partial(
        pl.kernel,
        out_type=jax.ShapeDtypeStruct((2 * _NP, HP), jnp.float32),
        mesh=mesh,
        compiler_params=pltpu.CompilerParams(needs_layout_passes=False,
                                             use_tc_tiling_on_sc=False),
        scratch_types=[
            pltpu.VMEM((2, _K), jnp.int32),
            pltpu.VMEM((2, 2, _KH), jnp.int32),
            pltpu.VMEM((2, _K, H), jnp.float32),
            pltpu.VMEM((2, _K, H // 2), jnp.int32),
            pltpu.VMEM((2, _K, H // 2), jnp.int32),
            pltpu.VMEM((2, _KH, HP), jnp.float32),
            pltpu.VMEM((H,), jnp.float32),
            pltpu.VMEM_SHARED((_NP, HP), jnp.float32),
            pltpu.SemaphoreType.DMA,
            pltpu.SemaphoreType.DMA,
            pltpu.SemaphoreType.DMA,
            pltpu.SemaphoreType.DMA,
            pltpu.SemaphoreType.DMA,
            pltpu.SemaphoreType.DMA,
            pltpu.SemaphoreType.DMA,
            pltpu.SemaphoreType.DMA,
        ],
    )
    def k(xl_hbm, xr_hbm, em_hbm, src_hbm, dst_hbm, att_hbm, out_hbm,
          src_v, dst_v, rows_l, rows_r, em_v, wout, att_v, acc_sh,
          sl0, sr0, se0, sl1, sr1, se1, ss0, ss1):
        c = lax.axis_index("c")
        s = lax.axis_index("s")
        wid = c * 16 + s
        sems = ((sl0, sr0, se0), (sl1, sr1, se1))
        ssem = (ss0, ss1)

        pltpu.sync_copy(att_hbm, att_v)

        zero16 = jnp.zeros((16,), jnp.float32)

        def zw(e, carry):
            for h in range(2):
                for j in range(HP // 16):
                    wout[h, e, pl.ds(j * 16, 16)] = zero16
            return carry

        lax.fori_loop(0, _KH, zw, 0)
        for t in range(_NPT // _KH):
            pltpu.sync_copy(wout.at[t % 2],
                            acc_sh.at[pl.ds(s * _NPT + t * _KH, _KH)])
        plsc.subcore_barrier()

        lane = lax.iota(jnp.int32, 16)

        def issue(ch, b):
            base = wid * _EPW + ch * _K
            pltpu.sync_copy(src_hbm.at[pl.ds(base, _K)], src_v.at[b])
            pltpu.sync_copy(dst_hbm.at[pl.ds(base // _KH, 2)], dst_v.at[b])
            pltpu.async_copy(xl_hbm.at[src_v.at[b]], rows_l.at[b], sems[b][0])
            for h in range(2):
                pltpu.async_copy(xr_hbm.at[dst_v.at[b, h]],
                                 rows_r.at[b, pl.ds(h * _KH, _KH)],
                                 sems[b][1])
            pltpu.async_copy(em_hbm.at[pl.ds(base, _K)], em_v.at[b],
                             sems[b][2])

        def drain(b):
            pltpu.make_async_copy(xl_hbm.at[src_v.at[b]], rows_l.at[b],
                                  sems[b][0]).wait()
            for h in range(2):
                pltpu.make_async_copy(xr_hbm.at[dst_v.at[b, h]],
                                      rows_r.at[b, pl.ds(h * _KH, _KH)],
                                      sems[b][1]).wait()
            pltpu.make_async_copy(em_hbm.at[pl.ds(0, _K)], em_v.at[b],
                                  sems[b][2]).wait()

        def scat_wait(b, h):
            pltpu.make_async_copy(wout.at[h],
                                  acc_sh.at[dst_v.at[b, h]],
                                  ssem[h]).wait()

        def compute_half(b, h, att_regs):
            rl = rows_l.at[b]
            rr = rows_r.at[b]
            ev = em_v.at[b]

            def group(g, regs):
                w0 = g * _U
                for u in range(_U):
                    e = h * _KH + w0 + u
                    w = w0 + u
                    acc = None
                    lvs = [None] * NG
                    for q in range(NG // 2):
                        sl = pl.ds(q * 16, 16)
                        wr = rr[e, sl]
                        we = ev[e, sl]
                        lo = (plsc.bitcast(wr << 16, jnp.float32)
                              + plsc.bitcast(we << 16, jnp.float32))
                        hi = (plsc.bitcast(wr & _HIMASK, jnp.float32)
                              + plsc.bitcast(we & _HIMASK, jnp.float32))
                        for j, addv in ((q, lo), (NG // 2 + q, hi)):
                            lv = rl[e, pl.ds(j * 16, 16)]
                            lvs[j] = lv
                            mv = lv + addv
                            mv = jnp.maximum(mv, 0.2 * mv)
                            t = mv * regs[j]
                            acc = t if acc is None else acc + t
                    a = jnp.sum(acc)
                    aev = jnp.exp(jnp.broadcast_to(a, (16,)))
                    for j in range(NG):
                        wout[h, w, pl.ds(j * 16, 16)] = lvs[j] * aev
                    wout[h, w, pl.ds(H, 16)] = jnp.where(lane == 0, aev, 0.0)
                return regs

            return lax.fori_loop(0, _KH // _U, group, att_regs)

        att_regs0 = tuple(att_v[pl.ds(j * 16, 16)] for j in range(NG))

        issue(0, 0)

        def outer(i, att_regs):
            for b in (0, 1):
                ch = i * 2 + b
                drain(b)

                @pl.when(ch + 1 < _NCH)
                def _():
                    issue(ch + 1, 1 - b)

                for h in (0, 1):
                    @pl.when(ch >= 1)
                    def _():
                        scat_wait(1 - b, h)

                    att_regs = compute_half(b, h, att_regs)
                    pltpu.async_copy(wout.at[h],
                                     acc_sh.at[dst_v.at[b, h]],
                                     ssem[h], add=True)
            return att_regs

        lax.fori_loop(0, _NCH // 2, outer, att_regs0)
        for h in (0, 1):
            scat_wait(1, h)
        plsc.subcore_barrier()

        pltpu.sync_copy(acc_sh.at[pl.ds(s * _NPT, _NPT)],
                        out_hbm.at[pl.ds(c * _NP + s * _NPT, _NPT)])

    return k


_sc_edge_h1 = _sc_edge_pass(_H1)
_sc_edge_h2 = _sc_edge_pass(_H2)


def _tc_mid_body(acc_ref, bias_ref, gamma_ref, beta_ref,
                 wl_ref, bl_ref, wr_ref, br_ref, xl_ref, xr_ref):
    accv = acc_ref[...]
    tot = accv[0:_N, :] + accv[_NP:_NP + _N, :]
    asum = tot[:, _H1:_H1 + 1]
    d = tot[:, 0:_H1] / (asum + 1e-16) + bias_ref[...]
    mu = jnp.mean(d, axis=0, keepdims=True)
    var = jnp.mean((d - mu) ** 2, axis=0, keepdims=True)
    dn = (d - mu) / jnp.sqrt(var + 1e-5) * gamma_ref[...] + beta_ref[...]
    d1 = jnp.maximum(dn, 0.01 * dn)
    xl_ref[...] = jnp.dot(d1, wl_ref[...],
                          preferred_element_type=jnp.float32) + bl_ref[...]
    xr = jnp.dot(d1, wr_ref[...],
                 preferred_element_type=jnp.float32) + br_ref[...]
    xr_ref[...] = _pack_cols(xr)


def _tc_mid(acc1, bias1, gamma1, beta1, W_l2, b_l2, W_r2, b_r2):
    return pl.pallas_call(
        _tc_mid_body,
        out_shape=(jax.ShapeDtypeStruct((_N, _H2), jnp.float32),
                   jax.ShapeDtypeStruct((_N, _H2 // 2), jnp.int32)),
    )(acc1, bias1, gamma1, beta1, W_l2, b_l2, W_r2, b_r2)


def _tc_final_body(acc_ref, bias_ref, gamma_ref, beta_ref,
                   gw_ref, gb_ref, batch_ref, out_ref):
    accv = acc_ref[...]
    tot = accv[0:_N, :] + accv[_NP:_NP + _N, :]
    asum = tot[:, _H2:_H2 + 1]
    d = tot[:, 0:_H2] / (asum + 1e-16) + bias_ref[...]
    mu = jnp.mean(d, axis=0, keepdims=True)
    var = jnp.mean((d - mu) ** 2, axis=0, keepdims=True)
    dn = (d - mu) / jnp.sqrt(var + 1e-5) * gamma_ref[...] + beta_ref[...]
    d2 = jnp.maximum(dn, 0.01 * dn)
    gate = jnp.dot(d2, gw_ref[...],
                   preferred_element_type=jnp.float32) + gb_ref[...]
    ge = jnp.exp(gate)
    oh = (lax.broadcasted_iota(jnp.int32, (_G, _N), 0)
          == batch_ref[...]).astype(jnp.float32)
    wsum = jnp.dot(oh, d2 * ge, preferred_element_type=jnp.float32)
    gs = jnp.dot(oh, ge, preferred_element_type=jnp.float32)
    out_ref[...] = wsum / (gs + 1e-16)


def _tc_final(acc2, bias2, gamma2, beta2, gate_W, gate_b, batch_row):
    return pl.pallas_call(
        _tc_final_body,
        out_shape=jax.ShapeDtypeStruct((_G, _H2), jnp.float32),
    )(acc2, bias2, gamma2, beta2, gate_W, gate_b, batch_row)


def kernel(x, edge_index, edge_attr, batch, emb_table, W_l1, b_l1, W_r1, b_r1,
           W_e1, att1, bias1, gamma1, beta1, W_l2, b_l2, W_r2, b_r2, W_e2,
           att2, bias2, gamma2, beta2, gate_W, gate_b):
    src = edge_index[0]
    dst = edge_index[1].reshape(_E // _KH, _KH)

    xl1, xr1 = _tc_node(x, emb_table,
                        W_l1, b_l1.reshape(1, _H1), W_r1, b_r1.reshape(1, _H1))
    em1, em2 = _tc_em(edge_attr, W_e1, W_e2)

    acc1 = _sc_edge_h1(xl1, xr1, em1, src, dst, att1)
    xl2, xr2 = _tc_mid(acc1, bias1.reshape(1, _H1), gamma1.reshape(1, _H1),
                       beta1.reshape(1, _H1), W_l2, b_l2.reshape(1, _H2),
                       W_r2, b_r2.reshape(1, _H2))

    acc2 = _sc_edge_h2(xl2, xr2, em2, src, dst, att2)
    out = _tc_final(acc2, bias2.reshape(1, _H2), gamma2.reshape(1, _H2),
                    beta2.reshape(1, _H2), gate_W, gate_b.reshape(1, 1),
                    batch.reshape(1, _N))
    return out
